# baseline (speedup 1.0000x reference)
import functools
import math

import jax
import jax.numpy as jnp
from jax import lax
from jax.experimental import pallas as pl
from jax.experimental.pallas import tpu as pltpu
from jax.experimental.pallas import tpu_sc as plsc

_LANES = 16


def _combo_tc_body(seg_ref, out_ref):
    R, D = out_ref.shape
    r = lax.broadcasted_iota(jnp.int32, (R, D), 0)
    dcol = lax.broadcasted_iota(jnp.int32, (R, D), 1)
    l3 = r // 3
    s = r - 3 * l3
    half = (dcol // 2).astype(jnp.float32)
    div = jnp.exp(half * (-2.0 * math.log(10000.0) / D))
    ang = l3.astype(jnp.float32) * div
    pe = jnp.where(dcol % 2 == 0, jnp.sin(ang), jnp.cos(ang))
    st = seg_ref[...]
    seg0 = jnp.broadcast_to(st[0:1, :], (R, D))
    seg1 = jnp.broadcast_to(st[1:2, :], (R, D))
    seg2 = jnp.broadcast_to(st[2:3, :], (R, D))
    out_ref[...] = pe + jnp.where(s == 0, seg0, jnp.where(s == 1, seg1, seg2))


def _build_combo(segment_table, rows):
    return pl.pallas_call(
        _combo_tc_body,
        out_shape=jax.ShapeDtypeStruct((rows, segment_table.shape[1]), jnp.float32),
    )(segment_table)


def _sc_lookup(seq_flat, lbl_flat, token_table, combo, L):
    N = seq_flat.shape[0]
    D = token_table.shape[1]
    info = plsc.get_sparse_core_info()
    NC, NS = info.num_cores, info.num_subcores
    NW = NC * NS
    C = 80
    assert N % (NW * C) == 0 and D % _LANES == 0
    rows_per_w = N // NW
    chunks = rows_per_w // C
    assert rows_per_w % L == 0 and C <= L
    NBUF = 8
    assert chunks % NBUF == 0
    mesh = plsc.VectorSubcoreMesh(core_axis_name="c", subcore_axis_name="s")

    @functools.partial(
        pl.kernel,
        out_type=jax.ShapeDtypeStruct((N, D), jnp.float32),
        mesh=mesh,
        scratch_types=(
            [pltpu.VMEM((chunks, C), jnp.int32)] * 2
            + [pltpu.VMEM((C, D), jnp.float32)] * NBUF
            + [pltpu.VMEM_SHARED((NS * ((3 * L + NS * 8 - 1) // (NS * 8)) * 8,
                                  D), jnp.float32)]
            + [pltpu.SemaphoreType.DMA] * (3 * NBUF + 1)
        ),
    )
    def k(seq_hbm, lbl_hbm, tok_hbm, combo_hbm, out_hbm, *sc):
        sidx_all, cidx_all = sc[0], sc[1]
        tok = sc[2:2 + NBUF]
        combo_spm = sc[2 + NBUF]
        sems = sc[3 + NBUF:]
        tsem, csem = sems[:NBUF], sems[NBUF:2 * NBUF]
        wsem, psem = sems[2 * NBUF:3 * NBUF], sems[3 * NBUF]
        wid = lax.axis_index("s") * NC + lax.axis_index("c")
        tile_base = wid * rows_per_w
        sid = lax.axis_index("s")
        rows_per_tile = combo_spm.shape[0] // NS
        stg = pltpu.async_copy(
            combo_hbm.at[pl.ds(sid * rows_per_tile, rows_per_tile)],
            combo_spm.at[pl.ds(sid * rows_per_tile, rows_per_tile)], psem)
        sg = pltpu.async_copy(seq_hbm.at[wid], sidx_all, tsem[0])
        cg0 = pltpu.async_copy(lbl_hbm.at[wid], cidx_all, csem[0])
        sg.wait()
        cg0.wait()

        def cvt(c, lpos0):
            for j in range(C // _LANES):
                v = lpos0 + (j * _LANES + lax.iota(jnp.int32, _LANES))
                lpos = jnp.where(v >= L, v - L, v)
                sl = pl.ds(j * _LANES, _LANES)
                cidx_all[c, sl] = 3 * lpos + cidx_all[c, sl]
            nxt = lpos0 + C
            return jnp.where(nxt >= L, nxt - L, nxt)

        lax.fori_loop(0, chunks, cvt, jnp.int32(0), unroll=False)
        stg.wait()
        plsc.subcore_barrier()

        def body(kk, carry):
            c0 = kk * NBUF
            tgs = []
            for b in range(NBUF):
                @pl.when(kk > 0)
                def _(b=b):
                    pltpu.make_async_copy(
                        tok[b], out_hbm.at[pl.ds(tile_base, C)], wsem[b]
                    ).wait()
                tgs.append(pltpu.async_copy(tok_hbm.at[sidx_all.at[c0 + b]],
                                            tok[b], tsem[b]))
            cgs = []
            for b in range(NBUF):
                tgs[b].wait()
                cgs.append(pltpu.async_copy(combo_spm.at[cidx_all.at[c0 + b]],
                                            tok[b], csem[b], add=True))
            for b in range(NBUF):
                cgs[b].wait()
                base = tile_base + (c0 + b) * C
                pltpu.async_copy(tok[b], out_hbm.at[pl.ds(base, C)], wsem[b])
            return carry

        lax.fori_loop(0, chunks // NBUF, body, jnp.int32(0), unroll=False)
        for b in range(NBUF):
            pltpu.make_async_copy(
                tok[b], out_hbm.at[pl.ds(tile_base, C)], wsem[b]
            ).wait()

    seq3 = seq_flat.reshape(NW, chunks, C)
    lbl3 = lbl_flat.reshape(NW, chunks, C)
    return k(seq3, lbl3, token_table, combo)


def kernel(sequence, segment_label, token_table, segment_table):
    B, L = sequence.shape
    D = token_table.shape[1]
    NS = plsc.get_sparse_core_info().num_subcores
    combo_rows = NS * ((3 * L + NS * 8 - 1) // (NS * 8)) * 8
    combo = _build_combo(segment_table, combo_rows)
    seq_flat = sequence.reshape(-1).astype(jnp.int32)
    lbl_flat = segment_label.reshape(-1).astype(jnp.int32)
    out = _sc_lookup(seq_flat, lbl_flat, token_table, combo, L)
    return out.reshape(B, L, D)


# speedup vs baseline: 1.0145x; 1.0145x over previous
import functools
import math

import jax
import jax.numpy as jnp
from jax import lax
from jax.experimental import pallas as pl
from jax.experimental.pallas import tpu as pltpu
from jax.experimental.pallas import tpu_sc as plsc

_LANES = 16


def _combo_tc_body(seg_ref, out_ref):
    R, D = out_ref.shape
    r = lax.broadcasted_iota(jnp.int32, (R, D), 0)
    dcol = lax.broadcasted_iota(jnp.int32, (R, D), 1)
    l3 = r // 3
    s = r - 3 * l3
    half = (dcol // 2).astype(jnp.float32)
    div = jnp.exp(half * (-2.0 * math.log(10000.0) / D))
    ang = l3.astype(jnp.float32) * div
    pe = jnp.where(dcol % 2 == 0, jnp.sin(ang), jnp.cos(ang))
    st = seg_ref[...]
    seg0 = jnp.broadcast_to(st[0:1, :], (R, D))
    seg1 = jnp.broadcast_to(st[1:2, :], (R, D))
    seg2 = jnp.broadcast_to(st[2:3, :], (R, D))
    out_ref[...] = pe + jnp.where(s == 0, seg0, jnp.where(s == 1, seg1, seg2))


def _build_combo(segment_table, rows):
    return pl.pallas_call(
        _combo_tc_body,
        out_shape=jax.ShapeDtypeStruct((rows, segment_table.shape[1]), jnp.float32),
    )(segment_table)


def _sc_lookup(seq_flat, lbl_flat, token_table, combo, L):
    N = seq_flat.shape[0]
    D = token_table.shape[1]
    info = plsc.get_sparse_core_info()
    NC, NS = info.num_cores, info.num_subcores
    NW = NC * NS
    C = 128
    assert N % (NW * C) == 0 and D % _LANES == 0
    rows_per_w = N // NW
    chunks = rows_per_w // C
    assert rows_per_w % L == 0 and C <= L
    NBUF = 5
    assert chunks % NBUF == 0
    mesh = plsc.VectorSubcoreMesh(core_axis_name="c", subcore_axis_name="s")

    @functools.partial(
        pl.kernel,
        out_type=jax.ShapeDtypeStruct((N, D), jnp.float32),
        mesh=mesh,
        scratch_types=(
            [pltpu.VMEM((chunks, C), jnp.int32)] * 2
            + [pltpu.VMEM((C, D), jnp.float32)] * NBUF
            + [pltpu.VMEM_SHARED((NS * ((3 * L + NS * 8 - 1) // (NS * 8)) * 8,
                                  D), jnp.float32)]
            + [pltpu.SemaphoreType.DMA] * (3 * NBUF + 1)
        ),
    )
    def k(seq_hbm, lbl_hbm, tok_hbm, combo_hbm, out_hbm, *sc):
        sidx_all, cidx_all = sc[0], sc[1]
        tok = sc[2:2 + NBUF]
        combo_spm = sc[2 + NBUF]
        sems = sc[3 + NBUF:]
        tsem, csem = sems[:NBUF], sems[NBUF:2 * NBUF]
        wsem, psem = sems[2 * NBUF:3 * NBUF], sems[3 * NBUF]
        wid = lax.axis_index("s") * NC + lax.axis_index("c")
        tile_base = wid * rows_per_w
        sid = lax.axis_index("s")
        rows_per_tile = combo_spm.shape[0] // NS
        stg = pltpu.async_copy(
            combo_hbm.at[pl.ds(sid * rows_per_tile, rows_per_tile)],
            combo_spm.at[pl.ds(sid * rows_per_tile, rows_per_tile)], psem)
        sg = pltpu.async_copy(seq_hbm.at[wid], sidx_all, tsem[0])
        cg0 = pltpu.async_copy(lbl_hbm.at[wid], cidx_all, csem[0])
        sg.wait()
        cg0.wait()

        def cvt(c, lpos0):
            for j in range(C // _LANES):
                v = lpos0 + (j * _LANES + lax.iota(jnp.int32, _LANES))
                lpos = jnp.where(v >= L, v - L, v)
                sl = pl.ds(j * _LANES, _LANES)
                cidx_all[c, sl] = 3 * lpos + cidx_all[c, sl]
            nxt = lpos0 + C
            return jnp.where(nxt >= L, nxt - L, nxt)

        lax.fori_loop(0, chunks, cvt, jnp.int32(0), unroll=False)
        stg.wait()
        plsc.subcore_barrier()

        def body(kk, carry):
            c0 = kk * NBUF
            tgs = []
            for b in range(NBUF):
                @pl.when(kk > 0)
                def _(b=b):
                    pltpu.make_async_copy(
                        tok[b], out_hbm.at[pl.ds(tile_base, C)], wsem[b]
                    ).wait()
                tgs.append(pltpu.async_copy(tok_hbm.at[sidx_all.at[c0 + b]],
                                            tok[b], tsem[b]))
            cgs = []
            for b in range(NBUF):
                tgs[b].wait()
                cgs.append(pltpu.async_copy(combo_spm.at[cidx_all.at[c0 + b]],
                                            tok[b], csem[b], add=True))
            for b in range(NBUF):
                cgs[b].wait()
                base = tile_base + (c0 + b) * C
                pltpu.async_copy(tok[b], out_hbm.at[pl.ds(base, C)], wsem[b])
            return carry

        lax.fori_loop(0, chunks // NBUF, body, jnp.int32(0), unroll=False)
        for b in range(NBUF):
            pltpu.make_async_copy(
                tok[b], out_hbm.at[pl.ds(tile_base, C)], wsem[b]
            ).wait()

    seq3 = seq_flat.reshape(NW, chunks, C)
    lbl3 = lbl_flat.reshape(NW, chunks, C)
    return k(seq3, lbl3, token_table, combo)


def kernel(sequence, segment_label, token_table, segment_table):
    B, L = sequence.shape
    D = token_table.shape[1]
    NS = plsc.get_sparse_core_info().num_subcores
    combo_rows = NS * ((3 * L + NS * 8 - 1) // (NS * 8)) * 8
    combo = _build_combo(segment_table, combo_rows)
    seq_flat = sequence.reshape(-1).astype(jnp.int32)
    lbl_flat = segment_label.reshape(-1).astype(jnp.int32)
    out = _sc_lookup(seq_flat, lbl_flat, token_table, combo, L)
    return out.reshape(B, L, D)
